# Initial kernel scaffold; baseline (speedup 1.0000x reference)
import jax
import jax.numpy as jnp
from jax.experimental import pallas as pl


def kernel(state_tensor, table):
    raise NotImplementedError("write your pallas kernel here")


# speedup vs baseline: 13.3628x; 13.3628x over previous
import functools

import jax
import jax.numpy as jnp
from jax import lax
from jax.experimental import pallas as pl
from jax.experimental.pallas import tpu as pltpu
from jax.experimental.pallas import tpu_sc as plsc

B = 16384
L = 200
D = 32
NC = 2
NS = 16
NW = NC * NS
ROWS_PER_W = B // NW
G = 8
CHUNKS = ROWS_PER_W // G
IDX_MINOR = 100
IDX_ROWS = G * L // IDX_MINOR
INV_L = 1.0 / L

_mesh = plsc.VectorSubcoreMesh(core_axis_name="c", subcore_axis_name="s")


@functools.partial(
    pl.kernel,
    out_type=jax.ShapeDtypeStruct((B, D), jnp.float32),
    mesh=_mesh,
    compiler_params=pltpu.CompilerParams(use_tc_tiling_on_sc=False),
    scratch_types=[
        pltpu.VMEM((IDX_ROWS, IDX_MINOR), jnp.int32),
        pltpu.VMEM((G * L, D), jnp.float32),
        pltpu.VMEM((G, D), jnp.float32),
        pltpu.SemaphoreType.DMA,
    ],
)
def _pooled_gather(table_hbm, idx_hbm, out_hbm, idx_v, rows_v, out_v, sem):
    wid = lax.axis_index("s") * NC + lax.axis_index("c")
    row_base = wid * ROWS_PER_W

    def chunk_body(c, _):
        out_base = pl.multiple_of(row_base + c * G, G)
        idx_row_base = pl.multiple_of(out_base * L // IDX_MINOR, IDX_ROWS)
        pltpu.sync_copy(idx_hbm.at[pl.ds(idx_row_base, IDX_ROWS)], idx_v)
        copies = []
        for j in range(IDX_ROWS):
            copies.append(
                pltpu.async_copy(
                    table_hbm.at[idx_v.at[j]],
                    rows_v.at[pl.ds(j * IDX_MINOR, IDX_MINOR)],
                    sem,
                )
            )
        for cp in copies:
            cp.wait()

        for g in range(G):
            def red_body(i, accs):
                a0, a1 = accs
                base = g * L + i * 8
                for r in range(8):
                    a0 = a0 + rows_v[base + r, pl.ds(0, 16)]
                    a1 = a1 + rows_v[base + r, pl.ds(16, 16)]
                return a0, a1

            zero = jnp.zeros((16,), jnp.float32)
            a0, a1 = lax.fori_loop(0, L // 8, red_body, (zero, zero))
            out_v[g, pl.ds(0, 16)] = a0 * INV_L
            out_v[g, pl.ds(16, 16)] = a1 * INV_L

        pltpu.sync_copy(out_v, out_hbm.at[pl.ds(out_base, G)])
        return ()

    lax.fori_loop(0, CHUNKS, chunk_body, ())


def kernel(state_tensor, table):
    idx = state_tensor.astype(jnp.int32).reshape(B * L // IDX_MINOR, IDX_MINOR)
    return _pooled_gather(table, idx)


# speedup vs baseline: 16.1285x; 1.2070x over previous
import functools

import jax
import jax.numpy as jnp
from jax import lax
from jax.experimental import pallas as pl
from jax.experimental.pallas import tpu as pltpu
from jax.experimental.pallas import tpu_sc as plsc

B = 16384
L = 200
D = 32
NC = 2
NS = 16
NW = NC * NS
ROWS_PER_W = B // NW
G = 8
CHUNKS = ROWS_PER_W // G
IDX_MINOR = 100
IDX_ROWS = G * L // IDX_MINOR
INV_L = 1.0 / L

_mesh = plsc.VectorSubcoreMesh(core_axis_name="c", subcore_axis_name="s")


@functools.partial(
    pl.kernel,
    out_type=jax.ShapeDtypeStruct((B, D), jnp.float32),
    mesh=_mesh,
    compiler_params=pltpu.CompilerParams(use_tc_tiling_on_sc=False),
    scratch_types=[
        pltpu.VMEM((2, IDX_ROWS, IDX_MINOR), jnp.int32),
        pltpu.VMEM((G * L, D), jnp.float32),
        pltpu.VMEM((G * L, D), jnp.float32),
        pltpu.VMEM((G, D), jnp.float32),
        pltpu.SemaphoreType.DMA,
        pltpu.SemaphoreType.DMA,
    ],
)
def _pooled_gather(table_hbm, idx_hbm, out_hbm, idx_v, rows0_v, rows1_v,
                   out_v, sem0, sem1):
    wid = lax.axis_index("s") * NC + lax.axis_index("c")
    row_base = wid * ROWS_PER_W
    rows_bufs = (rows0_v, rows1_v)
    sems = (sem0, sem1)

    def start(c, buf):
        out_base = pl.multiple_of(row_base + c * G, G)
        idx_row_base = pl.multiple_of(out_base * L // IDX_MINOR, IDX_ROWS)
        pltpu.sync_copy(idx_hbm.at[pl.ds(idx_row_base, IDX_ROWS)],
                        idx_v.at[buf])
        for j in range(IDX_ROWS):
            pltpu.async_copy(
                table_hbm.at[idx_v.at[buf, j]],
                rows_bufs[buf].at[pl.ds(j * IDX_MINOR, IDX_MINOR)],
                sems[buf],
            )

    def drain(buf):
        pltpu.make_async_copy(
            table_hbm.at[pl.ds(0, G * L)], rows_bufs[buf], sems[buf]
        ).wait()

    def reduce_store(c, buf):
        rows_v = rows_bufs[buf]
        out_base = pl.multiple_of(row_base + c * G, G)
        for g in range(G):
            def red_body(i, accs):
                a0, a1, a2, a3 = accs
                base = g * L + i * 8
                for r in range(0, 8, 2):
                    a0 = a0 + rows_v[base + r, pl.ds(0, 16)]
                    a1 = a1 + rows_v[base + r, pl.ds(16, 16)]
                    a2 = a2 + rows_v[base + r + 1, pl.ds(0, 16)]
                    a3 = a3 + rows_v[base + r + 1, pl.ds(16, 16)]
                return a0, a1, a2, a3

            zero = jnp.zeros((16,), jnp.float32)
            a0, a1, a2, a3 = lax.fori_loop(
                0, L // 8, red_body, (zero, zero, zero, zero))
            out_v[g, pl.ds(0, 16)] = (a0 + a2) * INV_L
            out_v[g, pl.ds(16, 16)] = (a1 + a3) * INV_L
        pltpu.sync_copy(out_v, out_hbm.at[pl.ds(out_base, G)])

    start(0, 0)

    def pair_body(i, _):
        c0 = i * 2
        c1 = c0 + 1
        start(c1, 1)
        drain(0)
        reduce_store(c0, 0)

        @pl.when(c1 + 1 < CHUNKS)
        def _():
            start(c1 + 1, 0)

        drain(1)
        reduce_store(c1, 1)
        return ()

    lax.fori_loop(0, CHUNKS // 2, pair_body, ())


def kernel(state_tensor, table):
    idx = state_tensor.astype(jnp.int32).reshape(B * L // IDX_MINOR, IDX_MINOR)
    return _pooled_gather(table, idx)


# speedup vs baseline: 16.3947x; 1.0165x over previous
import functools

import jax
import jax.numpy as jnp
from jax import lax
from jax.experimental import pallas as pl
from jax.experimental.pallas import tpu as pltpu
from jax.experimental.pallas import tpu_sc as plsc

B = 16384
L = 200
D = 32
NC = 2
NS = 16
NW = NC * NS
ROWS_PER_W = B // NW
G = 8
CHUNKS = ROWS_PER_W // G
IDX_SPLITS = ((0, 104), (104, 96))
INV_L = 1.0 / L

_mesh = plsc.VectorSubcoreMesh(core_axis_name="c", subcore_axis_name="s")


@functools.partial(
    pl.kernel,
    out_type=jax.ShapeDtypeStruct((B, D), jnp.float32),
    mesh=_mesh,
    compiler_params=pltpu.CompilerParams(use_tc_tiling_on_sc=False),
    scratch_types=[
        pltpu.VMEM((2, G, L), jnp.int32),
        pltpu.VMEM((G * L, D), jnp.float32),
        pltpu.VMEM((G * L, D), jnp.float32),
        pltpu.VMEM((G, D), jnp.float32),
        pltpu.SemaphoreType.DMA,
        pltpu.SemaphoreType.DMA,
    ],
)
def _pooled_gather(table_hbm, idx_hbm, out_hbm, idx_v, rows0_v, rows1_v,
                   out_v, sem0, sem1):
    wid = lax.axis_index("s") * NC + lax.axis_index("c")
    row_base = wid * ROWS_PER_W
    rows_bufs = (rows0_v, rows1_v)
    sems = (sem0, sem1)

    def start(c, buf):
        out_base = pl.multiple_of(row_base + c * G, G)
        pltpu.sync_copy(idx_hbm.at[pl.ds(out_base, G)], idx_v.at[buf])
        for g in range(G):
            for off, size in IDX_SPLITS:
                pltpu.async_copy(
                    table_hbm.at[idx_v.at[buf, g, pl.ds(off, size)]],
                    rows_bufs[buf].at[pl.ds(g * L + off, size)],
                    sems[buf],
                )

    def drain(buf):
        pltpu.make_async_copy(
            table_hbm.at[pl.ds(0, G * L)], rows_bufs[buf], sems[buf]
        ).wait()

    def reduce_store(c, buf):
        rows_v = rows_bufs[buf]
        out_base = pl.multiple_of(row_base + c * G, G)
        for g in range(G):
            def red_body(i, accs):
                a0, a1, a2, a3 = accs
                base = g * L + i * 8
                for r in range(0, 8, 2):
                    a0 = a0 + rows_v[base + r, pl.ds(0, 16)]
                    a1 = a1 + rows_v[base + r, pl.ds(16, 16)]
                    a2 = a2 + rows_v[base + r + 1, pl.ds(0, 16)]
                    a3 = a3 + rows_v[base + r + 1, pl.ds(16, 16)]
                return a0, a1, a2, a3

            zero = jnp.zeros((16,), jnp.float32)
            a0, a1, a2, a3 = lax.fori_loop(
                0, L // 8, red_body, (zero, zero, zero, zero))
            out_v[g, pl.ds(0, 16)] = (a0 + a2) * INV_L
            out_v[g, pl.ds(16, 16)] = (a1 + a3) * INV_L
        pltpu.sync_copy(out_v, out_hbm.at[pl.ds(out_base, G)])

    start(0, 0)

    def pair_body(i, _):
        c0 = i * 2
        c1 = c0 + 1
        start(c1, 1)
        drain(0)
        reduce_store(c0, 0)

        @pl.when(c1 + 1 < CHUNKS)
        def _():
            start(c1 + 1, 0)

        drain(1)
        reduce_store(c1, 1)
        return ()

    lax.fori_loop(0, CHUNKS // 2, pair_body, ())


def kernel(state_tensor, table):
    return _pooled_gather(table, state_tensor.astype(jnp.int32))


# speedup vs baseline: 16.4120x; 1.0011x over previous
import functools

import jax
import jax.numpy as jnp
from jax import lax
from jax.experimental import pallas as pl
from jax.experimental.pallas import tpu as pltpu
from jax.experimental.pallas import tpu_sc as plsc

B = 16384
L = 200
D = 32
NC = 2
NS = 16
NW = NC * NS
ROWS_PER_W = B // NW
G = 8
CHUNKS = ROWS_PER_W // G
IDX_SPLITS = ((0, 104), (104, 96))
INV_L = 1.0 / L

_mesh = plsc.VectorSubcoreMesh(core_axis_name="c", subcore_axis_name="s")


@functools.partial(
    pl.kernel,
    out_type=jax.ShapeDtypeStruct((B, D), jnp.float32),
    mesh=_mesh,
    compiler_params=pltpu.CompilerParams(use_tc_tiling_on_sc=False),
    scratch_types=[
        pltpu.VMEM((2, G, L), jnp.int32),
        pltpu.VMEM((G * L, D), jnp.float32),
        pltpu.VMEM((G * L, D), jnp.float32),
        pltpu.VMEM((G, D), jnp.float32),
        pltpu.SemaphoreType.DMA,
        pltpu.SemaphoreType.DMA,
    ],
)
def _pooled_gather(table_hbm, idx_hbm, out_hbm, idx_v, rows0_v, rows1_v,
                   out_v, sem0, sem1):
    wid = lax.axis_index("s") * NC + lax.axis_index("c")
    row_base = wid * ROWS_PER_W
    rows_bufs = (rows0_v, rows1_v)
    sems = (sem0, sem1)

    def start(c, buf):
        out_base = pl.multiple_of(row_base + c * G, G)
        pltpu.sync_copy(idx_hbm.at[pl.ds(out_base, G)], idx_v.at[buf])
        for g in range(G):
            for off, size in IDX_SPLITS:
                pltpu.async_copy(
                    table_hbm.at[idx_v.at[buf, g, pl.ds(off, size)]],
                    rows_bufs[buf].at[pl.ds(g * L + off, size)],
                    sems[buf],
                )

    def drain(buf):
        pltpu.make_async_copy(
            table_hbm.at[pl.ds(0, G * L)], rows_bufs[buf], sems[buf]
        ).wait()

    def reduce_store(c, buf):
        rows_v = rows_bufs[buf]
        out_base = pl.multiple_of(row_base + c * G, G)
        for g in range(G):
            def red_body(i, accs):
                a0, a1, a2, a3 = accs
                base = g * L + i * 8
                for r in range(0, 8, 2):
                    a0 = a0 + rows_v[base + r, pl.ds(0, 16)]
                    a1 = a1 + rows_v[base + r, pl.ds(16, 16)]
                    a2 = a2 + rows_v[base + r + 1, pl.ds(0, 16)]
                    a3 = a3 + rows_v[base + r + 1, pl.ds(16, 16)]
                return a0, a1, a2, a3

            zero = jnp.zeros((16,), jnp.float32)
            a0, a1, a2, a3 = lax.fori_loop(
                0, L // 8, red_body, (zero, zero, zero, zero))
            out_v[g, pl.ds(0, 16)] = (a0 + a2) * INV_L
            out_v[g, pl.ds(16, 16)] = (a1 + a3) * INV_L
        pltpu.sync_copy(out_v, out_hbm.at[pl.ds(out_base, G)])

    start(0, 0)

    def pair_body(i, _):
        c0 = i * 2
        c1 = c0 + 1
        start(c1, 1)
        drain(0)
        reduce_store(c0, 0)

        @pl.when(c1 + 1 < CHUNKS)
        def _():
            start(c1 + 1, 0)

        drain(1)
        reduce_store(c1, 1)
        return ()

    lax.fori_loop(0, CHUNKS // 2, pair_body, ())


def kernel(state_tensor, table):
    if state_tensor.dtype != jnp.int32:
        state_tensor = state_tensor.astype(jnp.int32)
    return _pooled_gather(table, state_tensor)
